# Initial kernel scaffold; baseline (speedup 1.0000x reference)
import jax
import jax.numpy as jnp
from jax.experimental import pallas as pl


def kernel(x, edge_index, Wq, bq, Wk, bk, Wv, bv, Wo, bo, W1, b1, W2, b2, g1, be1, g2, be2):
    raise NotImplementedError("write your pallas kernel here")


# speedup vs baseline: 27.6808x; 27.6808x over previous
import functools

import jax
import jax.numpy as jnp
from jax import lax
from jax.experimental import pallas as pl
from jax.experimental.pallas import tpu as pltpu
from jax.experimental.pallas import tpu_sc as plsc

_N = 10000
_E = 320000
_D = 128
_H = 8
_C = 16
_FF = 256

_NR = 10016
_K = 80
_HQ = 2
_DQ = _HQ * _C
_EPS = _E // 16
_NCH = _EPS // _K
_RPT = _NR // 16

_BLK = 1000
_NBLK = _N // _BLK


def _sc_edge_body(scores_hbm, src_hbm, dst_hbm, vh0, vh1, vh2, vh3,
                  zraw_hbm, zden_hbm,
                  raw_out, den_out,
                  sbuf, sidxb, didxb, dstb, exb, vbuf, msgb, raw_s, den_s,
                  sem):
    cid = lax.axis_index("c")
    sid = lax.axis_index("s")
    r0 = sid * _RPT
    iota = lax.broadcasted_iota(jnp.int32, (16,), 0)

    pltpu.sync_copy(scores_hbm, sbuf)
    zero = jnp.zeros((16,), jnp.float32)

    def zrow(i, c):
        plsc.store_scatter(exb, [jnp.full((16,), i, jnp.int32), iota], zero)
        return c

    lax.fori_loop(0, _K, zrow, 0)

    ebase = sid * _EPS

    def run(vh_ref, h0):
        def chunk(ci, carry):
            base = ebase + ci * _K
            pltpu.sync_copy(src_hbm.at[pl.ds(base, _K)], sidxb)
            pltpu.sync_copy(dst_hbm.at[pl.ds(base, _K)], didxb)
            gath = pltpu.async_copy(vh_ref.at[sidxb], vbuf, sem)

            for g in range(_K // 16):
                s_idx = sidxb[pl.ds(g * 16, 16)]
                d_idx = didxb[pl.ds(g * 16, 16)]
                dstb[pl.ds(g * 16, 16)] = jnp.where(
                    s_idx == d_idx, jnp.int32(_N), d_idx)
                rows = iota + g * 16
                for h in range(_HQ):
                    hvec = jnp.full((16,), h, jnp.int32) + h0
                    a = plsc.load_gather(sbuf, [s_idx, hvec])
                    b = plsc.load_gather(sbuf, [d_idx, hvec])
                    t = a + b
                    t = jnp.maximum(t, 0.2 * t)
                    plsc.store_scatter(
                        exb, [rows, jnp.full((16,), h, jnp.int32)],
                        jnp.exp(t))

            gath.wait()

            def edge(i, carry2):
                row0 = jnp.full((16,), i, jnp.int32)
                erep = plsc.load_gather(
                    exb, [row0, jnp.bitwise_and(iota, _HQ - 1)])
                for w in range(_DQ // 16):
                    cols = iota + w * 16
                    v = plsc.load_gather(vbuf, [row0, cols])
                    plsc.store_scatter(msgb, [row0, cols], v * erep)
                return carry2

            lax.fori_loop(0, _K, edge, 0)

            pltpu.sync_copy(exb, den_s.at[dstb], add=True)
            pltpu.sync_copy(msgb, raw_s.at[dstb], add=True)
            return carry

        lax.fori_loop(0, _NCH, chunk, 0)

    for p in range(2):
        pltpu.sync_copy(zraw_hbm.at[pl.ds(r0, _RPT)],
                        raw_s.at[pl.ds(r0, _RPT)])
        pltpu.sync_copy(zden_hbm.at[pl.ds(r0, _RPT)],
                        den_s.at[pl.ds(r0, _RPT)])
        plsc.subcore_barrier()

        @pl.when(cid == 0)
        def _():
            run((vh0, vh2)[p], jnp.int32(4 * p))

        @pl.when(cid == 1)
        def _():
            run((vh1, vh3)[p], jnp.int32(4 * p + 2))

        plsc.subcore_barrier()
        q = 2 * p + cid
        pltpu.sync_copy(raw_s.at[pl.ds(r0, _RPT)],
                        raw_out.at[q, pl.ds(r0, _RPT)])
        pltpu.sync_copy(den_s.at[pl.ds(r0, _RPT)],
                        den_out.at[q, pl.ds(r0, _RPT)])


@functools.cache
def _sc_edge_call():
    return pl.kernel(
        _sc_edge_body,
    out_type=(
        jax.ShapeDtypeStruct((4, _NR, _DQ), jnp.float32),
        jax.ShapeDtypeStruct((4, _NR, 16), jnp.float32),
    ),
        mesh=plsc.VectorSubcoreMesh(core_axis_name="c", subcore_axis_name="s"),
        compiler_params=pltpu.CompilerParams(
            needs_layout_passes=False, use_tc_tiling_on_sc=False),
        scratch_types=[
            pltpu.VMEM((_N, _H), jnp.float32),
            pltpu.VMEM((_K,), jnp.int32),
            pltpu.VMEM((_K,), jnp.int32),
            pltpu.VMEM((_K,), jnp.int32),
            pltpu.VMEM((_K, 16), jnp.float32),
            pltpu.VMEM((_K, _DQ), jnp.float32),
            pltpu.VMEM((_K, _DQ), jnp.float32),
            pltpu.VMEM_SHARED((_NR, _DQ), jnp.float32),
            pltpu.VMEM_SHARED((_NR, 16), jnp.float32),
            pltpu.SemaphoreType.DMA,
        ],
    )


def _proj_body(x_ref, wqt, bq, wkt, bk, wvt, bv, mh, p,
               scores_ref, vh_ref, rawi_ref, deni_ref):
    xb = x_ref[...]
    q = jnp.dot(xb, wqt[...], preferred_element_type=jnp.float32) + bq[...]
    k = jnp.dot(xb, wkt[...], preferred_element_type=jnp.float32) + bk[...]
    v = jnp.dot(xb, wvt[...], preferred_element_type=jnp.float32) + bv[...]
    s = jnp.dot(q * k, mh[...], preferred_element_type=jnp.float32) * 0.25
    scores_ref[...] = s
    vh_ref[...] = v
    e0 = jnp.exp(jnp.maximum(2.0 * s, 0.4 * s))
    deni_ref[...] = e0
    rawi_ref[...] = jnp.dot(e0, p[...], preferred_element_type=jnp.float32) * v


def _agg_body(rq0, rq1, rq2, rq3, rawi, dq0, dq1, dq2, dq3, deni,
              x_ref, wot, bo, p2m, y_ref, st_ref):
    i = pl.program_id(0)
    f32 = jnp.float32
    den_all = deni[...] + jnp.concatenate(
        [dq0[...][:, :_HQ], dq1[...][:, :_HQ],
         dq2[...][:, :_HQ], dq3[...][:, :_HQ]], axis=1)
    denb = jnp.dot(den_all, p2m[...], preferred_element_type=f32) + 1e-16
    raw = jnp.concatenate([rq0[...], rq1[...], rq2[...], rq3[...]], axis=1)
    t = (raw + rawi[...]) / denb
    y = jnp.dot(t, wot[...], preferred_element_type=f32)
    y = y + bo[...] + x_ref[...]
    y_ref[...] = y

    @pl.when(i == 0)
    def _():
        st_ref[...] = jnp.zeros_like(st_ref)

    st_ref[0:1, :] += jnp.sum(y, axis=0, keepdims=True)
    st_ref[1:2, :] += jnp.sum(y * y, axis=0, keepdims=True)


def _ffn_body(y_ref, st_ref, g1, be1, w1t, b1, w2t, b2, f_ref, st2_ref):
    i = pl.program_id(0)
    mean = st_ref[0:1, :] * (1.0 / _N)
    var = st_ref[1:2, :] * (1.0 / _N) - mean * mean
    h = (y_ref[...] - mean) * lax.rsqrt(var + 1e-5) * g1[...] + be1[...]
    t = jnp.dot(h, w1t[...], preferred_element_type=jnp.float32) + b1[...]
    t = jnp.maximum(t, 0.0)
    f = jnp.dot(t, w2t[...], preferred_element_type=jnp.float32) + b2[...] + h
    f_ref[...] = f

    @pl.when(i == 0)
    def _():
        st2_ref[...] = jnp.zeros_like(st2_ref)

    st2_ref[0:1, :] += jnp.sum(f, axis=0, keepdims=True)
    st2_ref[1:2, :] += jnp.sum(f * f, axis=0, keepdims=True)


def _bn2_body(f_ref, st2_ref, g2, be2, out_ref):
    mean = st2_ref[0:1, :] * (1.0 / _N)
    var = st2_ref[1:2, :] * (1.0 / _N) - mean * mean
    out_ref[...] = ((f_ref[...] - mean) * lax.rsqrt(var + 1e-5)
                    * g2[...] + be2[...])


def _full(spec):
    return pl.BlockSpec(spec, lambda i: tuple(0 for _ in spec))


def _rows(w):
    return pl.BlockSpec((_BLK, w), lambda i: (i, 0))


def kernel(x, edge_index, Wq, bq, Wk, bk, Wv, bv, Wo, bo,
           W1, b1, W2, b2, g1, be1, g2, be2):
    f32 = jnp.float32
    j = jnp.arange(_D)
    hd = _HQ * (j // _DQ) + j % _HQ
    perm = hd * _C + (j % _DQ) // _HQ
    hsel = jnp.arange(_H)
    mh = (j[:, None] // _C == hsel[None, :]).astype(f32)
    p2 = (hd[None, :] == hsel[:, None]).astype(f32)

    wqt = Wq.T
    wkt = Wk.T
    wvt = Wv.T[:, perm]
    bvp = bv[perm].reshape(1, _D)
    wot = Wo[:, perm].T
    bq2 = bq.reshape(1, _D)
    bk2 = bk.reshape(1, _D)
    bo2 = bo.reshape(1, _D)
    b12 = b1.reshape(1, _FF)
    b22 = b2.reshape(1, _D)
    g12 = g1.reshape(1, _D)
    be12 = be1.reshape(1, _D)
    g22 = g2.reshape(1, _D)
    be22 = be2.reshape(1, _D)

    scores, vh, rawi, deni = pl.pallas_call(
        _proj_body,
        grid=(_NBLK,),
        in_specs=[
            _rows(_D),
            _full((_D, _D)), _full((1, _D)),
            _full((_D, _D)), _full((1, _D)),
            _full((_D, _D)), _full((1, _D)),
            _full((_D, _H)), _full((_H, _D)),
        ],
        out_specs=[_rows(_H), _rows(_D), _rows(_D), _rows(_H)],
        out_shape=[
            jax.ShapeDtypeStruct((_N, _H), f32),
            jax.ShapeDtypeStruct((_N, _D), f32),
            jax.ShapeDtypeStruct((_N, _D), f32),
            jax.ShapeDtypeStruct((_N, _H), f32),
        ],
    )(x, wqt, bq2, wkt, bk2, wvt, bvp, mh, p2)

    zraw = jnp.zeros((_NR, _DQ), f32)
    zden = jnp.zeros((_NR, 16), f32)
    raw_sc, den_sc = _sc_edge_call()(
        scores, edge_index[0], edge_index[1],
        vh[:, 0 * _DQ:1 * _DQ], vh[:, 1 * _DQ:2 * _DQ],
        vh[:, 2 * _DQ:3 * _DQ], vh[:, 3 * _DQ:4 * _DQ], zraw, zden)

    y, st = pl.pallas_call(
        _agg_body,
        grid=(_NBLK,),
        in_specs=[
            _rows(_DQ), _rows(_DQ), _rows(_DQ), _rows(_DQ), _rows(_D),
            _rows(16), _rows(16), _rows(16), _rows(16), _rows(_H),
            _rows(_D), _full((_D, _D)), _full((1, _D)), _full((_H, _D)),
        ],
        out_specs=[_rows(_D), _full((8, _D))],
        out_shape=[
            jax.ShapeDtypeStruct((_N, _D), f32),
            jax.ShapeDtypeStruct((8, _D), f32),
        ],
    )(raw_sc[0, :_N], raw_sc[1, :_N], raw_sc[2, :_N], raw_sc[3, :_N], rawi,
      den_sc[0, :_N], den_sc[1, :_N], den_sc[2, :_N], den_sc[3, :_N], deni,
      x, wot, bo2, p2)

    f, st2 = pl.pallas_call(
        _ffn_body,
        grid=(_NBLK,),
        in_specs=[
            _rows(_D), _full((8, _D)), _full((1, _D)), _full((1, _D)),
            _full((_D, _FF)), _full((1, _FF)),
            _full((_FF, _D)), _full((1, _D)),
        ],
        out_specs=[_rows(_D), _full((8, _D))],
        out_shape=[
            jax.ShapeDtypeStruct((_N, _D), f32),
            jax.ShapeDtypeStruct((8, _D), f32),
        ],
    )(y, st, g12, be12, W1.T, b12, W2.T, b22)

    out = pl.pallas_call(
        _bn2_body,
        grid=(_NBLK,),
        in_specs=[_rows(_D), _full((8, _D)), _full((1, _D)), _full((1, _D))],
        out_specs=_rows(_D),
        out_shape=jax.ShapeDtypeStruct((_N, _D), f32),
    )(f, st2, g22, be22)

    return out


# speedup vs baseline: 32.1224x; 1.1605x over previous
import functools

import jax
import jax.numpy as jnp
from jax import lax
from jax.experimental import pallas as pl
from jax.experimental.pallas import tpu as pltpu
from jax.experimental.pallas import tpu_sc as plsc

_N = 10000
_E = 320000
_D = 128
_H = 8
_C = 16
_FF = 256

_NR = 10016
_K = 80
_HQ = 2
_DQ = _HQ * _C
_EPS = _E // 16
_NCH = _EPS // _K
_RPT = _NR // 16

_BLK = 1000
_NBLK = _N // _BLK


def _sc_edge_body(scores_hbm, src_hbm, dst_hbm, vh0, vh1, vh2, vh3,
                  zraw_hbm, zden_hbm,
                  raw_out, den_out,
                  sbuf,
                  sidxb0, sidxb1, didxb0, didxb1, dstb0, dstb1,
                  exb0, exb1, vbuf0, vbuf1,
                  raw_s, den_s, sem0, sem1):
    cid = lax.axis_index("c")
    sid = lax.axis_index("s")
    r0 = sid * _RPT
    iota = lax.broadcasted_iota(jnp.int32, (16,), 0)
    sidxb = (sidxb0, sidxb1)
    didxb = (didxb0, didxb1)
    dstb = (dstb0, dstb1)
    exb = (exb0, exb1)
    vbuf = (vbuf0, vbuf1)
    sem = (sem0, sem1)
    ebase = sid * _EPS

    zero = jnp.zeros((16,), jnp.float32)

    def zrow(i, c):
        row = jnp.full((16,), i, jnp.int32)
        plsc.store_scatter(exb0, [row, iota], zero)
        plsc.store_scatter(exb1, [row, iota], zero)
        return c

    lax.fori_loop(0, _K, zrow, 0)

    def run(vh_ref):
        def fetch(ci, par):
            base = ebase + ci * _K
            pltpu.sync_copy(src_hbm.at[pl.ds(base, _K)], sidxb[par])
            pltpu.sync_copy(dst_hbm.at[pl.ds(base, _K)], didxb[par])

        def gather(par):
            pltpu.async_copy(vh_ref.at[sidxb[par]], vbuf[par], sem[par])

        def prep(par):
            for g in range(_K // 16):
                s_idx = sidxb[par][pl.ds(g * 16, 16)]
                d_idx = didxb[par][pl.ds(g * 16, 16)]
                dstb[par][pl.ds(g * 16, 16)] = jnp.where(
                    s_idx == d_idx, jnp.int32(_N), d_idx)
                rows = iota + g * 16
                for h in range(_HQ):
                    hvec = jnp.full((16,), h, jnp.int32)
                    a = plsc.load_gather(sbuf, [s_idx, hvec])
                    b = plsc.load_gather(sbuf, [d_idx, hvec])
                    t = a + b
                    t = jnp.maximum(t, 0.2 * t)
                    plsc.store_scatter(exb[par], [rows, hvec], jnp.exp(t))

        def proc(par):
            pltpu.make_async_copy(
                vh_ref.at[sidxb[par]], vbuf[par], sem[par]).wait()

            def edge(it, carry2):
                for u in range(4):
                    row0 = jnp.full((16,), it * 4 + u, jnp.int32)
                    erep = plsc.load_gather(
                        exb[par], [row0, jnp.bitwise_and(iota, _HQ - 1)])
                    for w in range(_DQ // 16):
                        cols = iota + w * 16
                        v = plsc.load_gather(vbuf[par], [row0, cols])
                        plsc.store_scatter(vbuf[par], [row0, cols], v * erep)
                return carry2

            lax.fori_loop(0, _K // 4, edge, 0)
            pltpu.sync_copy(exb[par], den_s.at[dstb[par]], add=True)
            pltpu.sync_copy(vbuf[par], raw_s.at[dstb[par]], add=True)

        fetch(0, 0)
        gather(0)

        def pipe(ci2, carry):
            for par in range(2):
                ci = 2 * ci2 + par
                nxt = jnp.minimum(ci + 1, _NCH - 1)
                fetch(nxt, 1 - par)
                gather(1 - par)
                prep(par)
                proc(par)
            return carry

        lax.fori_loop(0, _NCH // 2, pipe, 0)
        pltpu.make_async_copy(vh_ref.at[sidxb[0]], vbuf[0], sem[0]).wait()

    for p in range(2):
        pltpu.sync_copy(zraw_hbm.at[pl.ds(r0, _RPT)],
                        raw_s.at[pl.ds(r0, _RPT)])
        pltpu.sync_copy(zden_hbm.at[pl.ds(r0, _RPT)],
                        den_s.at[pl.ds(r0, _RPT)])
        q = 2 * p + cid
        pltpu.sync_copy(scores_hbm.at[q], sbuf)
        plsc.subcore_barrier()

        @pl.when(cid == 0)
        def _():
            run((vh0, vh2)[p])

        @pl.when(cid == 1)
        def _():
            run((vh1, vh3)[p])

        plsc.subcore_barrier()
        pltpu.sync_copy(raw_s.at[pl.ds(r0, _RPT)],
                        raw_out.at[q, pl.ds(r0, _RPT)])
        pltpu.sync_copy(den_s.at[pl.ds(r0, _RPT)],
                        den_out.at[q, pl.ds(r0, _RPT)])


@functools.cache
def _sc_edge_call():
    return pl.kernel(
        _sc_edge_body,
    out_type=(
        jax.ShapeDtypeStruct((4, _NR, _DQ), jnp.float32),
        jax.ShapeDtypeStruct((4, _NR, 16), jnp.float32),
    ),
        mesh=plsc.VectorSubcoreMesh(core_axis_name="c", subcore_axis_name="s"),
        compiler_params=pltpu.CompilerParams(
            needs_layout_passes=False, use_tc_tiling_on_sc=False),
        scratch_types=[
            pltpu.VMEM((_N, _HQ), jnp.float32),
            pltpu.VMEM((_K,), jnp.int32),
            pltpu.VMEM((_K,), jnp.int32),
            pltpu.VMEM((_K,), jnp.int32),
            pltpu.VMEM((_K,), jnp.int32),
            pltpu.VMEM((_K,), jnp.int32),
            pltpu.VMEM((_K,), jnp.int32),
            pltpu.VMEM((_K, 16), jnp.float32),
            pltpu.VMEM((_K, 16), jnp.float32),
            pltpu.VMEM((_K, _DQ), jnp.float32),
            pltpu.VMEM((_K, _DQ), jnp.float32),
            pltpu.VMEM_SHARED((_NR, _DQ), jnp.float32),
            pltpu.VMEM_SHARED((_NR, 16), jnp.float32),
            pltpu.SemaphoreType.DMA,
            pltpu.SemaphoreType.DMA,
        ],
    )


def _proj_body(x_ref, wqt, bq, wkt, bk, wvt, bv, mh, p,
               scores_ref, vh_ref, rawi_ref, deni_ref):
    xb = x_ref[...]
    q = jnp.dot(xb, wqt[...], preferred_element_type=jnp.float32) + bq[...]
    k = jnp.dot(xb, wkt[...], preferred_element_type=jnp.float32) + bk[...]
    v = jnp.dot(xb, wvt[...], preferred_element_type=jnp.float32) + bv[...]
    s = jnp.dot(q * k, mh[...], preferred_element_type=jnp.float32) * 0.25
    scores_ref[...] = s
    vh_ref[...] = v
    e0 = jnp.exp(jnp.maximum(2.0 * s, 0.4 * s))
    deni_ref[...] = e0
    rawi_ref[...] = jnp.dot(e0, p[...], preferred_element_type=jnp.float32) * v


def _agg_body(rq0, rq1, rq2, rq3, rawi, dq0, dq1, dq2, dq3, deni,
              x_ref, wot, bo, p2m, y_ref, st_ref):
    i = pl.program_id(0)
    f32 = jnp.float32
    den_all = deni[...] + jnp.concatenate(
        [dq0[...][:, :_HQ], dq1[...][:, :_HQ],
         dq2[...][:, :_HQ], dq3[...][:, :_HQ]], axis=1)
    denb = jnp.dot(den_all, p2m[...], preferred_element_type=f32) + 1e-16
    raw = jnp.concatenate([rq0[...], rq1[...], rq2[...], rq3[...]], axis=1)
    t = (raw + rawi[...]) / denb
    y = jnp.dot(t, wot[...], preferred_element_type=f32)
    y = y + bo[...] + x_ref[...]
    y_ref[...] = y

    @pl.when(i == 0)
    def _():
        st_ref[...] = jnp.zeros_like(st_ref)

    st_ref[0:1, :] += jnp.sum(y, axis=0, keepdims=True)
    st_ref[1:2, :] += jnp.sum(y * y, axis=0, keepdims=True)


def _ffn_body(y_ref, st_ref, g1, be1, w1t, b1, w2t, b2, f_ref, st2_ref):
    i = pl.program_id(0)
    mean = st_ref[0:1, :] * (1.0 / _N)
    var = st_ref[1:2, :] * (1.0 / _N) - mean * mean
    h = (y_ref[...] - mean) * lax.rsqrt(var + 1e-5) * g1[...] + be1[...]
    t = jnp.dot(h, w1t[...], preferred_element_type=jnp.float32) + b1[...]
    t = jnp.maximum(t, 0.0)
    f = jnp.dot(t, w2t[...], preferred_element_type=jnp.float32) + b2[...] + h
    f_ref[...] = f

    @pl.when(i == 0)
    def _():
        st2_ref[...] = jnp.zeros_like(st2_ref)

    st2_ref[0:1, :] += jnp.sum(f, axis=0, keepdims=True)
    st2_ref[1:2, :] += jnp.sum(f * f, axis=0, keepdims=True)


def _bn2_body(f_ref, st2_ref, g2, be2, out_ref):
    mean = st2_ref[0:1, :] * (1.0 / _N)
    var = st2_ref[1:2, :] * (1.0 / _N) - mean * mean
    out_ref[...] = ((f_ref[...] - mean) * lax.rsqrt(var + 1e-5)
                    * g2[...] + be2[...])


def _full(spec):
    return pl.BlockSpec(spec, lambda i: tuple(0 for _ in spec))


def _rows(w):
    return pl.BlockSpec((_BLK, w), lambda i: (i, 0))


def kernel(x, edge_index, Wq, bq, Wk, bk, Wv, bv, Wo, bo,
           W1, b1, W2, b2, g1, be1, g2, be2):
    f32 = jnp.float32
    j = jnp.arange(_D)
    hd = _HQ * (j // _DQ) + j % _HQ
    perm = hd * _C + (j % _DQ) // _HQ
    hsel = jnp.arange(_H)
    mh = (j[:, None] // _C == hsel[None, :]).astype(f32)
    p2 = (hd[None, :] == hsel[:, None]).astype(f32)

    wqt = Wq.T
    wkt = Wk.T
    wvt = Wv.T[:, perm]
    bvp = bv[perm].reshape(1, _D)
    wot = Wo[:, perm].T
    bq2 = bq.reshape(1, _D)
    bk2 = bk.reshape(1, _D)
    bo2 = bo.reshape(1, _D)
    b12 = b1.reshape(1, _FF)
    b22 = b2.reshape(1, _D)
    g12 = g1.reshape(1, _D)
    be12 = be1.reshape(1, _D)
    g22 = g2.reshape(1, _D)
    be22 = be2.reshape(1, _D)

    scores, vh, rawi, deni = pl.pallas_call(
        _proj_body,
        grid=(_NBLK,),
        in_specs=[
            _rows(_D),
            _full((_D, _D)), _full((1, _D)),
            _full((_D, _D)), _full((1, _D)),
            _full((_D, _D)), _full((1, _D)),
            _full((_D, _H)), _full((_H, _D)),
        ],
        out_specs=[_rows(_H), _rows(_D), _rows(_D), _rows(_H)],
        out_shape=[
            jax.ShapeDtypeStruct((_N, _H), f32),
            jax.ShapeDtypeStruct((_N, _D), f32),
            jax.ShapeDtypeStruct((_N, _D), f32),
            jax.ShapeDtypeStruct((_N, _H), f32),
        ],
    )(x, wqt, bq2, wkt, bk2, wvt, bvp, mh, p2)

    zraw = jnp.zeros((_NR, _DQ), f32)
    zden = jnp.zeros((_NR, 16), f32)
    s4 = jnp.transpose(scores.reshape(_N, 4, _HQ), (1, 0, 2))
    raw_sc, den_sc = _sc_edge_call()(
        s4, edge_index[0], edge_index[1],
        vh[:, 0 * _DQ:1 * _DQ], vh[:, 1 * _DQ:2 * _DQ],
        vh[:, 2 * _DQ:3 * _DQ], vh[:, 3 * _DQ:4 * _DQ], zraw, zden)

    y, st = pl.pallas_call(
        _agg_body,
        grid=(_NBLK,),
        in_specs=[
            _rows(_DQ), _rows(_DQ), _rows(_DQ), _rows(_DQ), _rows(_D),
            _rows(16), _rows(16), _rows(16), _rows(16), _rows(_H),
            _rows(_D), _full((_D, _D)), _full((1, _D)), _full((_H, _D)),
        ],
        out_specs=[_rows(_D), _full((8, _D))],
        out_shape=[
            jax.ShapeDtypeStruct((_N, _D), f32),
            jax.ShapeDtypeStruct((8, _D), f32),
        ],
    )(raw_sc[0, :_N], raw_sc[1, :_N], raw_sc[2, :_N], raw_sc[3, :_N], rawi,
      den_sc[0, :_N], den_sc[1, :_N], den_sc[2, :_N], den_sc[3, :_N], deni,
      x, wot, bo2, p2)

    f, st2 = pl.pallas_call(
        _ffn_body,
        grid=(_NBLK,),
        in_specs=[
            _rows(_D), _full((8, _D)), _full((1, _D)), _full((1, _D)),
            _full((_D, _FF)), _full((1, _FF)),
            _full((_FF, _D)), _full((1, _D)),
        ],
        out_specs=[_rows(_D), _full((8, _D))],
        out_shape=[
            jax.ShapeDtypeStruct((_N, _D), f32),
            jax.ShapeDtypeStruct((8, _D), f32),
        ],
    )(y, st, g12, be12, W1.T, b12, W2.T, b22)

    out = pl.pallas_call(
        _bn2_body,
        grid=(_NBLK,),
        in_specs=[_rows(_D), _full((8, _D)), _full((1, _D)), _full((1, _D))],
        out_specs=_rows(_D),
        out_shape=jax.ShapeDtypeStruct((_N, _D), f32),
    )(f, st2, g22, be22)

    return out


# speedup vs baseline: 38.7919x; 1.2076x over previous
import functools

import jax
import jax.numpy as jnp
from jax import lax
from jax.experimental import pallas as pl
from jax.experimental.pallas import tpu as pltpu
from jax.experimental.pallas import tpu_sc as plsc

_N = 10000
_E = 320000
_D = 128
_H = 8
_C = 16
_FF = 256

_NR = 10016
_K = 80
_HQ = 2
_DQ = _HQ * _C
_EPS = _E // 16
_NCH = _EPS // _K
_RPT = _NR // 16

_BLK = 1000
_NBLK = _N // _BLK


def _sc_edge_body(scores_hbm, epack_hbm, vh0, vh1, vh2, vh3, zacc_hbm,
                  acc_out,
                  sbuf, ebuf0, ebuf1, vbuf0, vbuf1, vvb0, vvb1,
                  acc_s, semg0, semg1, sems0, sems1):
    cid = lax.axis_index("c")
    sid = lax.axis_index("s")
    r0 = sid * _RPT
    iota = lax.broadcasted_iota(jnp.int32, (16,), 0)
    ebuf = (ebuf0, ebuf1)
    vbuf = (vbuf0, vbuf1)
    vvb = (vvb0, vvb1)
    semg = (semg0, semg1)
    sems = (sems0, sems1)

    zero = jnp.zeros((16,), jnp.float32)

    def zrow(i, c):
        row = jnp.full((16,), i, jnp.int32)
        plsc.store_scatter(vvb0, [row, iota + _DQ], zero)
        plsc.store_scatter(vvb1, [row, iota + _DQ], zero)
        return c

    lax.fori_loop(0, _K, zrow, 0)

    def run(vh_ref):
        def fetch(ci, par):
            pltpu.sync_copy(epack_hbm.at[sid * _NCH + ci], ebuf[par])

        def gather(par):
            pltpu.async_copy(
                vh_ref.at[ebuf[par].at[0]], vbuf[par], semg[par])

        def drain_scatter(par):
            pltpu.make_async_copy(
                vvb[par], acc_s.at[ebuf[par].at[1]], sems[par]).wait()

        def prep(par):
            for g in range(_K // 16):
                cols16 = iota + g * 16
                s_idx = plsc.load_gather(ebuf[par], [jnp.zeros(
                    (16,), jnp.int32), cols16])
                d_eff = plsc.load_gather(ebuf[par], [jnp.full(
                    (16,), 1, jnp.int32), cols16])
                d_sc = jnp.minimum(d_eff, jnp.int32(_N - 1))
                for h in range(_HQ):
                    hvec = jnp.full((16,), h, jnp.int32)
                    a = plsc.load_gather(sbuf, [s_idx, hvec])
                    b = plsc.load_gather(sbuf, [d_sc, hvec])
                    t = a + b
                    t = jnp.maximum(t, 0.2 * t)
                    plsc.store_scatter(
                        vvb[par], [cols16, hvec + _DQ], jnp.exp(t))

        def proc(par):
            pltpu.make_async_copy(
                vh_ref.at[ebuf[par].at[0]], vbuf[par], semg[par]).wait()

            def edge(it, carry2):
                for u in range(4):
                    row0 = jnp.full((16,), it * 4 + u, jnp.int32)
                    erep = plsc.load_gather(
                        vvb[par],
                        [row0, jnp.bitwise_and(iota, _HQ - 1) + _DQ])
                    for w in range(_DQ // 16):
                        cols = iota + w * 16
                        v = plsc.load_gather(vbuf[par], [row0, cols])
                        plsc.store_scatter(vvb[par], [row0, cols], v * erep)
                return carry2

            lax.fori_loop(0, _K // 4, edge, 0)
            pltpu.async_copy(
                vvb[par], acc_s.at[ebuf[par].at[1]], sems[par], add=True)

        fetch(0, 0)
        gather(0)

        def pipe(ci2, carry):
            for par in range(2):
                ci = 2 * ci2 + par
                if par == 0:
                    @pl.when(ci2 > 0)
                    def _():
                        drain_scatter(1)
                else:
                    drain_scatter(0)
                nxt = jnp.minimum(ci + 1, _NCH - 1)
                fetch(nxt, 1 - par)
                gather(1 - par)
                prep(par)
                proc(par)
            return carry

        lax.fori_loop(0, _NCH // 2, pipe, 0)
        drain_scatter(1)
        pltpu.make_async_copy(
            vh_ref.at[ebuf[0].at[0]], vbuf[0], semg[0]).wait()

    for p in range(2):
        pltpu.sync_copy(zacc_hbm.at[pl.ds(r0, _RPT)],
                        acc_s.at[pl.ds(r0, _RPT)])
        q = 2 * p + cid
        pltpu.sync_copy(scores_hbm.at[q], sbuf)
        plsc.subcore_barrier()

        @pl.when(cid == 0)
        def _():
            run((vh0, vh2)[p])

        @pl.when(cid == 1)
        def _():
            run((vh1, vh3)[p])

        plsc.subcore_barrier()
        pltpu.sync_copy(acc_s.at[pl.ds(r0, _RPT)],
                        acc_out.at[q, pl.ds(r0, _RPT)])


@functools.cache
def _sc_edge_call():
    return pl.kernel(
        _sc_edge_body,
    out_type=jax.ShapeDtypeStruct((4, _NR, 48), jnp.float32),
        mesh=plsc.VectorSubcoreMesh(core_axis_name="c", subcore_axis_name="s"),
        compiler_params=pltpu.CompilerParams(
            needs_layout_passes=False, use_tc_tiling_on_sc=False),
        scratch_types=[
            pltpu.VMEM((_N, _HQ), jnp.float32),
            pltpu.VMEM((2, _K), jnp.int32),
            pltpu.VMEM((2, _K), jnp.int32),
            pltpu.VMEM((_K, _DQ), jnp.float32),
            pltpu.VMEM((_K, _DQ), jnp.float32),
            pltpu.VMEM((_K, 48), jnp.float32),
            pltpu.VMEM((_K, 48), jnp.float32),
            pltpu.VMEM_SHARED((_NR, 48), jnp.float32),
            pltpu.SemaphoreType.DMA,
            pltpu.SemaphoreType.DMA,
            pltpu.SemaphoreType.DMA,
            pltpu.SemaphoreType.DMA,
        ],
    )


def _proj_body(x_ref, wqt, bq, wkt, bk, wvt, bv, mh, p,
               scores_ref, vh_ref, rawi_ref, deni_ref):
    xb = x_ref[...]
    q = jnp.dot(xb, wqt[...], preferred_element_type=jnp.float32) + bq[...]
    k = jnp.dot(xb, wkt[...], preferred_element_type=jnp.float32) + bk[...]
    v = jnp.dot(xb, wvt[...], preferred_element_type=jnp.float32) + bv[...]
    s = jnp.dot(q * k, mh[...], preferred_element_type=jnp.float32) * 0.25
    scores_ref[...] = s
    vh_ref[...] = v
    e0 = jnp.exp(jnp.maximum(2.0 * s, 0.4 * s))
    deni_ref[...] = e0
    rawi_ref[...] = jnp.dot(e0, p[...], preferred_element_type=jnp.float32) * v


def _agg_body(aq0, aq1, aq2, aq3, rawi, deni,
              x_ref, wot, bo, p2m, y_ref, st_ref):
    i = pl.program_id(0)
    f32 = jnp.float32
    a0, a1, a2, a3 = aq0[...], aq1[...], aq2[...], aq3[...]
    den_all = deni[...] + jnp.concatenate(
        [a0[:, _DQ:_DQ + _HQ], a1[:, _DQ:_DQ + _HQ],
         a2[:, _DQ:_DQ + _HQ], a3[:, _DQ:_DQ + _HQ]], axis=1)
    denb = jnp.dot(den_all, p2m[...], preferred_element_type=f32) + 1e-16
    raw = jnp.concatenate(
        [a0[:, :_DQ], a1[:, :_DQ], a2[:, :_DQ], a3[:, :_DQ]], axis=1)
    t = (raw + rawi[...]) / denb
    y = jnp.dot(t, wot[...], preferred_element_type=f32)
    y = y + bo[...] + x_ref[...]
    y_ref[...] = y

    @pl.when(i == 0)
    def _():
        st_ref[...] = jnp.zeros_like(st_ref)

    st_ref[0:1, :] += jnp.sum(y, axis=0, keepdims=True)
    st_ref[1:2, :] += jnp.sum(y * y, axis=0, keepdims=True)


def _ffn_body(y_ref, st_ref, g1, be1, w1t, b1, w2t, b2, f_ref, st2_ref):
    i = pl.program_id(0)
    mean = st_ref[0:1, :] * (1.0 / _N)
    var = st_ref[1:2, :] * (1.0 / _N) - mean * mean
    h = (y_ref[...] - mean) * lax.rsqrt(var + 1e-5) * g1[...] + be1[...]
    t = jnp.dot(h, w1t[...], preferred_element_type=jnp.float32) + b1[...]
    t = jnp.maximum(t, 0.0)
    f = jnp.dot(t, w2t[...], preferred_element_type=jnp.float32) + b2[...] + h
    f_ref[...] = f

    @pl.when(i == 0)
    def _():
        st2_ref[...] = jnp.zeros_like(st2_ref)

    st2_ref[0:1, :] += jnp.sum(f, axis=0, keepdims=True)
    st2_ref[1:2, :] += jnp.sum(f * f, axis=0, keepdims=True)


def _bn2_body(f_ref, st2_ref, g2, be2, out_ref):
    mean = st2_ref[0:1, :] * (1.0 / _N)
    var = st2_ref[1:2, :] * (1.0 / _N) - mean * mean
    out_ref[...] = ((f_ref[...] - mean) * lax.rsqrt(var + 1e-5)
                    * g2[...] + be2[...])


def _full(spec):
    return pl.BlockSpec(spec, lambda i: tuple(0 for _ in spec))


def _rows(w):
    return pl.BlockSpec((_BLK, w), lambda i: (i, 0))


def kernel(x, edge_index, Wq, bq, Wk, bk, Wv, bv, Wo, bo,
           W1, b1, W2, b2, g1, be1, g2, be2):
    f32 = jnp.float32
    j = jnp.arange(_D)
    hd = _HQ * (j // _DQ) + j % _HQ
    perm = hd * _C + (j % _DQ) // _HQ
    hsel = jnp.arange(_H)
    mh = (j[:, None] // _C == hsel[None, :]).astype(f32)
    p2 = (hd[None, :] == hsel[:, None]).astype(f32)

    wqt = Wq.T
    wkt = Wk.T
    wvt = Wv.T[:, perm]
    bvp = bv[perm].reshape(1, _D)
    wot = Wo[:, perm].T
    bq2 = bq.reshape(1, _D)
    bk2 = bk.reshape(1, _D)
    bo2 = bo.reshape(1, _D)
    b12 = b1.reshape(1, _FF)
    b22 = b2.reshape(1, _D)
    g12 = g1.reshape(1, _D)
    be12 = be1.reshape(1, _D)
    g22 = g2.reshape(1, _D)
    be22 = be2.reshape(1, _D)

    scores, vh, rawi, deni = pl.pallas_call(
        _proj_body,
        grid=(_NBLK,),
        in_specs=[
            _rows(_D),
            _full((_D, _D)), _full((1, _D)),
            _full((_D, _D)), _full((1, _D)),
            _full((_D, _D)), _full((1, _D)),
            _full((_D, _H)), _full((_H, _D)),
        ],
        out_specs=[_rows(_H), _rows(_D), _rows(_D), _rows(_H)],
        out_shape=[
            jax.ShapeDtypeStruct((_N, _H), f32),
            jax.ShapeDtypeStruct((_N, _D), f32),
            jax.ShapeDtypeStruct((_N, _D), f32),
            jax.ShapeDtypeStruct((_N, _H), f32),
        ],
    )(x, wqt, bq2, wkt, bk2, wvt, bvp, mh, p2)

    zacc = jnp.zeros((_NR, 48), f32)
    s4 = jnp.transpose(scores.reshape(_N, 4, _HQ), (1, 0, 2))
    src = edge_index[0]
    dst = edge_index[1]
    deff = jnp.where(src == dst, jnp.int32(_N), dst)
    epack = jnp.stack(
        [src.reshape(-1, _K), deff.reshape(-1, _K)], axis=1)
    acc = _sc_edge_call()(
        s4, epack,
        vh[:, 0 * _DQ:1 * _DQ], vh[:, 1 * _DQ:2 * _DQ],
        vh[:, 2 * _DQ:3 * _DQ], vh[:, 3 * _DQ:4 * _DQ], zacc)

    y, st = pl.pallas_call(
        _agg_body,
        grid=(_NBLK,),
        in_specs=[
            _rows(48), _rows(48), _rows(48), _rows(48),
            _rows(_D), _rows(_H),
            _rows(_D), _full((_D, _D)), _full((1, _D)), _full((_H, _D)),
        ],
        out_specs=[_rows(_D), _full((8, _D))],
        out_shape=[
            jax.ShapeDtypeStruct((_N, _D), f32),
            jax.ShapeDtypeStruct((8, _D), f32),
        ],
    )(acc[0, :_N], acc[1, :_N], acc[2, :_N], acc[3, :_N], rawi, deni,
      x, wot, bo2, p2)

    f, st2 = pl.pallas_call(
        _ffn_body,
        grid=(_NBLK,),
        in_specs=[
            _rows(_D), _full((8, _D)), _full((1, _D)), _full((1, _D)),
            _full((_D, _FF)), _full((1, _FF)),
            _full((_FF, _D)), _full((1, _D)),
        ],
        out_specs=[_rows(_D), _full((8, _D))],
        out_shape=[
            jax.ShapeDtypeStruct((_N, _D), f32),
            jax.ShapeDtypeStruct((8, _D), f32),
        ],
    )(y, st, g12, be12, W1.T, b12, W2.T, b22)

    out = pl.pallas_call(
        _bn2_body,
        grid=(_NBLK,),
        in_specs=[_rows(_D), _full((8, _D)), _full((1, _D)), _full((1, _D))],
        out_specs=_rows(_D),
        out_shape=jax.ShapeDtypeStruct((_N, _D), f32),
    )(f, st2, g22, be22)

    return out


# speedup vs baseline: 39.9199x; 1.0291x over previous
import functools

import jax
import jax.numpy as jnp
from jax import lax
from jax.experimental import pallas as pl
from jax.experimental.pallas import tpu as pltpu
from jax.experimental.pallas import tpu_sc as plsc

_N = 10000
_E = 320000
_D = 128
_H = 8
_C = 16
_FF = 256

_NR = 10016
_K = 80
_HQ = 2
_DQ = _HQ * _C
_EPS = _E // 16
_NCH = _EPS // _K
_RPT = _NR // 16

_BLK = 1000
_NBLK = _N // _BLK


def _sc_edge_body(scores_hbm, epack_hbm, vh0, vh1, vh2, vh3, zacc_hbm,
                  acc_out,
                  sbuf, ebuf0, ebuf1, vbuf0, vbuf1, vvb0, vvb1,
                  acc_s, semg0, semg1, sems0, sems1):
    cid = lax.axis_index("c")
    sid = lax.axis_index("s")
    r0 = sid * _RPT
    iota = lax.broadcasted_iota(jnp.int32, (16,), 0)
    ebuf = (ebuf0, ebuf1)
    vbuf = (vbuf0, vbuf1)
    vvb = (vvb0, vvb1)
    semg = (semg0, semg1)
    sems = (sems0, sems1)

    zero = jnp.zeros((16,), jnp.float32)

    def zrow(i, c):
        row = jnp.full((16,), i, jnp.int32)
        plsc.store_scatter(vvb0, [row, iota + _DQ], zero)
        plsc.store_scatter(vvb1, [row, iota + _DQ], zero)
        return c

    lax.fori_loop(0, _K, zrow, 0)

    def run(vh_ref):
        def fetch(ci, par):
            pltpu.sync_copy(epack_hbm.at[sid * _NCH + ci], ebuf[par])

        def gather(par):
            pltpu.async_copy(
                vh_ref.at[ebuf[par].at[0]], vbuf[par], semg[par])

        def drain_scatter(par):
            pltpu.make_async_copy(
                vvb[par], acc_s.at[ebuf[par].at[1]], sems[par]).wait()

        def prep(par):
            for g in range(_K // 16):
                cols16 = iota + g * 16
                s_idx = plsc.load_gather(ebuf[par], [jnp.zeros(
                    (16,), jnp.int32), cols16])
                d_eff = plsc.load_gather(ebuf[par], [jnp.full(
                    (16,), 1, jnp.int32), cols16])
                d_sc = jnp.minimum(d_eff, jnp.int32(_N - 1))
                for h in range(_HQ):
                    hvec = jnp.full((16,), h, jnp.int32)
                    a = plsc.load_gather(sbuf, [s_idx, hvec])
                    b = plsc.load_gather(sbuf, [d_sc, hvec])
                    t = a + b
                    t = jnp.maximum(t, 0.2 * t)
                    plsc.store_scatter(
                        vvb[par], [cols16, hvec + _DQ], jnp.exp(t))

        def proc(par):
            pltpu.make_async_copy(
                vh_ref.at[ebuf[par].at[0]], vbuf[par], semg[par]).wait()

            def edge(it, carry2):
                for u in range(4):
                    i = it * 4 + u
                    row0 = jnp.full((16,), i, jnp.int32)
                    erep = plsc.load_gather(
                        vvb[par],
                        [row0, jnp.bitwise_and(iota, _HQ - 1) + _DQ])
                    for w in range(_DQ // 16):
                        v = vbuf[par][i, pl.ds(w * 16, 16)]
                        vvb[par][i, pl.ds(w * 16, 16)] = v * erep
                return carry2

            lax.fori_loop(0, _K // 4, edge, 0)
            pltpu.async_copy(
                vvb[par], acc_s.at[ebuf[par].at[1]], sems[par], add=True)

        fetch(0, 0)
        gather(0)

        def pipe(ci2, carry):
            for par in range(2):
                ci = 2 * ci2 + par
                if par == 0:
                    @pl.when(ci2 > 0)
                    def _():
                        drain_scatter(1)
                else:
                    drain_scatter(0)
                nxt = jnp.minimum(ci + 1, _NCH - 1)
                fetch(nxt, 1 - par)
                gather(1 - par)
                prep(par)
                proc(par)
            return carry

        lax.fori_loop(0, _NCH // 2, pipe, 0)
        drain_scatter(1)
        pltpu.make_async_copy(
            vh_ref.at[ebuf[0].at[0]], vbuf[0], semg[0]).wait()

    for p in range(2):
        pltpu.sync_copy(zacc_hbm.at[pl.ds(r0, _RPT)],
                        acc_s.at[pl.ds(r0, _RPT)])
        q = 2 * p + cid
        pltpu.sync_copy(scores_hbm.at[q], sbuf)
        plsc.subcore_barrier()

        @pl.when(cid == 0)
        def _():
            run((vh0, vh2)[p])

        @pl.when(cid == 1)
        def _():
            run((vh1, vh3)[p])

        plsc.subcore_barrier()
        pltpu.sync_copy(acc_s.at[pl.ds(r0, _RPT)],
                        acc_out.at[q, pl.ds(r0, _RPT)])


@functools.cache
def _sc_edge_call():
    return pl.kernel(
        _sc_edge_body,
    out_type=jax.ShapeDtypeStruct((4, _NR, 48), jnp.float32),
        mesh=plsc.VectorSubcoreMesh(core_axis_name="c", subcore_axis_name="s"),
        compiler_params=pltpu.CompilerParams(
            needs_layout_passes=False, use_tc_tiling_on_sc=False),
        scratch_types=[
            pltpu.VMEM((_N, _HQ), jnp.float32),
            pltpu.VMEM((2, _K), jnp.int32),
            pltpu.VMEM((2, _K), jnp.int32),
            pltpu.VMEM((_K, _DQ), jnp.float32),
            pltpu.VMEM((_K, _DQ), jnp.float32),
            pltpu.VMEM((_K, 48), jnp.float32),
            pltpu.VMEM((_K, 48), jnp.float32),
            pltpu.VMEM_SHARED((_NR, 48), jnp.float32),
            pltpu.SemaphoreType.DMA,
            pltpu.SemaphoreType.DMA,
            pltpu.SemaphoreType.DMA,
            pltpu.SemaphoreType.DMA,
        ],
    )


def _proj_body(x_ref, wqt, bq, wkt, bk, wvt, bv, mh, p,
               scores_ref, vh_ref, rawi_ref, deni_ref):
    xb = x_ref[...]
    q = jnp.dot(xb, wqt[...], preferred_element_type=jnp.float32) + bq[...]
    k = jnp.dot(xb, wkt[...], preferred_element_type=jnp.float32) + bk[...]
    v = jnp.dot(xb, wvt[...], preferred_element_type=jnp.float32) + bv[...]
    s = jnp.dot(q * k, mh[...], preferred_element_type=jnp.float32) * 0.25
    scores_ref[...] = s
    vh_ref[...] = v
    e0 = jnp.exp(jnp.maximum(2.0 * s, 0.4 * s))
    deni_ref[...] = e0
    rawi_ref[...] = jnp.dot(e0, p[...], preferred_element_type=jnp.float32) * v


def _agg_body(aq0, aq1, aq2, aq3, rawi, deni,
              x_ref, wot, bo, p2m, y_ref, st_ref):
    i = pl.program_id(0)
    f32 = jnp.float32
    a0, a1, a2, a3 = aq0[...], aq1[...], aq2[...], aq3[...]
    den_all = deni[...] + jnp.concatenate(
        [a0[:, _DQ:_DQ + _HQ], a1[:, _DQ:_DQ + _HQ],
         a2[:, _DQ:_DQ + _HQ], a3[:, _DQ:_DQ + _HQ]], axis=1)
    denb = jnp.dot(den_all, p2m[...], preferred_element_type=f32) + 1e-16
    raw = jnp.concatenate(
        [a0[:, :_DQ], a1[:, :_DQ], a2[:, :_DQ], a3[:, :_DQ]], axis=1)
    t = (raw + rawi[...]) / denb
    y = jnp.dot(t, wot[...], preferred_element_type=f32)
    y = y + bo[...] + x_ref[...]
    y_ref[...] = y

    @pl.when(i == 0)
    def _():
        st_ref[...] = jnp.zeros_like(st_ref)

    st_ref[0:1, :] += jnp.sum(y, axis=0, keepdims=True)
    st_ref[1:2, :] += jnp.sum(y * y, axis=0, keepdims=True)


def _ffn_body(y_ref, st_ref, g1, be1, w1t, b1, w2t, b2, f_ref, st2_ref):
    i = pl.program_id(0)
    mean = st_ref[0:1, :] * (1.0 / _N)
    var = st_ref[1:2, :] * (1.0 / _N) - mean * mean
    h = (y_ref[...] - mean) * lax.rsqrt(var + 1e-5) * g1[...] + be1[...]
    t = jnp.dot(h, w1t[...], preferred_element_type=jnp.float32) + b1[...]
    t = jnp.maximum(t, 0.0)
    f = jnp.dot(t, w2t[...], preferred_element_type=jnp.float32) + b2[...] + h
    f_ref[...] = f

    @pl.when(i == 0)
    def _():
        st2_ref[...] = jnp.zeros_like(st2_ref)

    st2_ref[0:1, :] += jnp.sum(f, axis=0, keepdims=True)
    st2_ref[1:2, :] += jnp.sum(f * f, axis=0, keepdims=True)


def _bn2_body(f_ref, st2_ref, g2, be2, out_ref):
    mean = st2_ref[0:1, :] * (1.0 / _N)
    var = st2_ref[1:2, :] * (1.0 / _N) - mean * mean
    out_ref[...] = ((f_ref[...] - mean) * lax.rsqrt(var + 1e-5)
                    * g2[...] + be2[...])


def _full(spec):
    return pl.BlockSpec(spec, lambda i: tuple(0 for _ in spec))


def _rows(w):
    return pl.BlockSpec((_BLK, w), lambda i: (i, 0))


def kernel(x, edge_index, Wq, bq, Wk, bk, Wv, bv, Wo, bo,
           W1, b1, W2, b2, g1, be1, g2, be2):
    f32 = jnp.float32
    j = jnp.arange(_D)
    hd = _HQ * (j // _DQ) + j % _HQ
    perm = hd * _C + (j % _DQ) // _HQ
    hsel = jnp.arange(_H)
    mh = (j[:, None] // _C == hsel[None, :]).astype(f32)
    p2 = (hd[None, :] == hsel[:, None]).astype(f32)

    wqt = Wq.T
    wkt = Wk.T
    wvt = Wv.T[:, perm]
    bvp = bv[perm].reshape(1, _D)
    wot = Wo[:, perm].T
    bq2 = bq.reshape(1, _D)
    bk2 = bk.reshape(1, _D)
    bo2 = bo.reshape(1, _D)
    b12 = b1.reshape(1, _FF)
    b22 = b2.reshape(1, _D)
    g12 = g1.reshape(1, _D)
    be12 = be1.reshape(1, _D)
    g22 = g2.reshape(1, _D)
    be22 = be2.reshape(1, _D)

    scores, vh, rawi, deni = pl.pallas_call(
        _proj_body,
        grid=(_NBLK,),
        in_specs=[
            _rows(_D),
            _full((_D, _D)), _full((1, _D)),
            _full((_D, _D)), _full((1, _D)),
            _full((_D, _D)), _full((1, _D)),
            _full((_D, _H)), _full((_H, _D)),
        ],
        out_specs=[_rows(_H), _rows(_D), _rows(_D), _rows(_H)],
        out_shape=[
            jax.ShapeDtypeStruct((_N, _H), f32),
            jax.ShapeDtypeStruct((_N, _D), f32),
            jax.ShapeDtypeStruct((_N, _D), f32),
            jax.ShapeDtypeStruct((_N, _H), f32),
        ],
    )(x, wqt, bq2, wkt, bk2, wvt, bvp, mh, p2)

    zacc = jnp.zeros((_NR, 48), f32)
    s4 = jnp.transpose(scores.reshape(_N, 4, _HQ), (1, 0, 2))
    src = edge_index[0]
    dst = edge_index[1]
    deff = jnp.where(src == dst, jnp.int32(_N), dst)
    epack = jnp.stack(
        [src.reshape(-1, _K), deff.reshape(-1, _K)], axis=1)
    acc = _sc_edge_call()(
        s4, epack,
        vh[:, 0 * _DQ:1 * _DQ], vh[:, 1 * _DQ:2 * _DQ],
        vh[:, 2 * _DQ:3 * _DQ], vh[:, 3 * _DQ:4 * _DQ], zacc)

    y, st = pl.pallas_call(
        _agg_body,
        grid=(_NBLK,),
        in_specs=[
            _rows(48), _rows(48), _rows(48), _rows(48),
            _rows(_D), _rows(_H),
            _rows(_D), _full((_D, _D)), _full((1, _D)), _full((_H, _D)),
        ],
        out_specs=[_rows(_D), _full((8, _D))],
        out_shape=[
            jax.ShapeDtypeStruct((_N, _D), f32),
            jax.ShapeDtypeStruct((8, _D), f32),
        ],
    )(acc[0, :_N], acc[1, :_N], acc[2, :_N], acc[3, :_N], rawi, deni,
      x, wot, bo2, p2)

    f, st2 = pl.pallas_call(
        _ffn_body,
        grid=(_NBLK,),
        in_specs=[
            _rows(_D), _full((8, _D)), _full((1, _D)), _full((1, _D)),
            _full((_D, _FF)), _full((1, _FF)),
            _full((_FF, _D)), _full((1, _D)),
        ],
        out_specs=[_rows(_D), _full((8, _D))],
        out_shape=[
            jax.ShapeDtypeStruct((_N, _D), f32),
            jax.ShapeDtypeStruct((8, _D), f32),
        ],
    )(y, st, g12, be12, W1.T, b12, W2.T, b22)

    out = pl.pallas_call(
        _bn2_body,
        grid=(_NBLK,),
        in_specs=[_rows(_D), _full((8, _D)), _full((1, _D)), _full((1, _D))],
        out_specs=_rows(_D),
        out_shape=jax.ShapeDtypeStruct((_N, _D), f32),
    )(f, st2, g22, be22)

    return out


# speedup vs baseline: 40.1158x; 1.0049x over previous
import functools

import jax
import jax.numpy as jnp
from jax import lax
from jax.experimental import pallas as pl
from jax.experimental.pallas import tpu as pltpu
from jax.experimental.pallas import tpu_sc as plsc

_N = 10000
_E = 320000
_D = 128
_H = 8
_C = 16
_FF = 256

_NR = 10016
_K = 80
_HQ = 2
_DQ = _HQ * _C
_EPS = _E // 16
_NCH = _EPS // _K
_RPT = _NR // 16

_BLK = 1000
_NBLK = _N // _BLK


def _sc_edge_body(scores_hbm, epack_hbm, vh0, vh1, vh2, vh3, zacc_hbm,
                  acc_out,
                  sbuf, ebuf0, ebuf1, vbuf0, vbuf1, vvb0, vvb1,
                  acc_s, semg0, semg1, sems0, sems1):
    cid = lax.axis_index("c")
    sid = lax.axis_index("s")
    r0 = sid * _RPT
    iota = lax.broadcasted_iota(jnp.int32, (16,), 0)
    ebuf = (ebuf0, ebuf1)
    vbuf = (vbuf0, vbuf1)
    vvb = (vvb0, vvb1)
    semg = (semg0, semg1)
    sems = (sems0, sems1)

    zero = jnp.zeros((16,), jnp.float32)

    def zrow(i, c):
        row = jnp.full((16,), i, jnp.int32)
        plsc.store_scatter(vvb0, [row, iota + _DQ], zero)
        plsc.store_scatter(vvb1, [row, iota + _DQ], zero)
        return c

    lax.fori_loop(0, _K, zrow, 0)

    def run(vh_ref):
        def fetch(ci, par):
            pltpu.sync_copy(epack_hbm.at[sid * _NCH + ci], ebuf[par])

        def gather(par):
            pltpu.async_copy(
                vh_ref.at[ebuf[par].at[0]], vbuf[par], semg[par])

        def drain_scatter(par):
            pltpu.make_async_copy(
                vvb[par], acc_s.at[ebuf[par].at[1]], sems[par]).wait()

        def prep(par):
            for g in range(_K // 16):
                cols16 = iota + g * 16
                s_idx = plsc.load_gather(ebuf[par], [jnp.zeros(
                    (16,), jnp.int32), cols16])
                d_eff = plsc.load_gather(ebuf[par], [jnp.full(
                    (16,), 1, jnp.int32), cols16])
                d_sc = jnp.minimum(d_eff, jnp.int32(_N - 1))
                for h in range(_HQ):
                    hvec = jnp.full((16,), h, jnp.int32)
                    a = plsc.load_gather(sbuf, [s_idx, hvec])
                    b = plsc.load_gather(sbuf, [d_sc, hvec])
                    t = a + b
                    t = jnp.maximum(t, 0.2 * t)
                    plsc.store_scatter(
                        vvb[par], [cols16, hvec + _DQ], jnp.exp(t))

        def proc(par):
            pltpu.make_async_copy(
                vh_ref.at[ebuf[par].at[0]], vbuf[par], semg[par]).wait()

            def edge(it, carry2):
                for u in range(8):
                    i = it * 8 + u
                    row0 = jnp.full((16,), i, jnp.int32)
                    erep = plsc.load_gather(
                        vvb[par],
                        [row0, jnp.bitwise_and(iota, _HQ - 1) + _DQ])
                    for w in range(_DQ // 16):
                        v = vbuf[par][i, pl.ds(w * 16, 16)]
                        vvb[par][i, pl.ds(w * 16, 16)] = v * erep
                return carry2

            lax.fori_loop(0, _K // 8, edge, 0)
            pltpu.async_copy(
                vvb[par], acc_s.at[ebuf[par].at[1]], sems[par], add=True)

        fetch(0, 0)
        gather(0)

        def pipe(ci2, carry):
            for par in range(2):
                ci = 2 * ci2 + par
                if par == 0:
                    @pl.when(ci2 > 0)
                    def _():
                        drain_scatter(1)
                else:
                    drain_scatter(0)
                nxt = jnp.minimum(ci + 1, _NCH - 1)
                fetch(nxt, 1 - par)
                gather(1 - par)
                prep(par)
                proc(par)
            return carry

        lax.fori_loop(0, _NCH // 2, pipe, 0)
        drain_scatter(1)
        pltpu.make_async_copy(
            vh_ref.at[ebuf[0].at[0]], vbuf[0], semg[0]).wait()

    for p in range(2):
        pltpu.sync_copy(zacc_hbm.at[pl.ds(r0, _RPT)],
                        acc_s.at[pl.ds(r0, _RPT)])
        q = 2 * p + cid
        pltpu.sync_copy(scores_hbm.at[q], sbuf)
        plsc.subcore_barrier()

        @pl.when(cid == 0)
        def _():
            run((vh0, vh2)[p])

        @pl.when(cid == 1)
        def _():
            run((vh1, vh3)[p])

        plsc.subcore_barrier()
        pltpu.sync_copy(acc_s.at[pl.ds(r0, _RPT)],
                        acc_out.at[q, pl.ds(r0, _RPT)])


@functools.cache
def _sc_edge_call():
    return pl.kernel(
        _sc_edge_body,
    out_type=jax.ShapeDtypeStruct((4, _NR, 48), jnp.float32),
        mesh=plsc.VectorSubcoreMesh(core_axis_name="c", subcore_axis_name="s"),
        compiler_params=pltpu.CompilerParams(
            needs_layout_passes=False, use_tc_tiling_on_sc=False),
        scratch_types=[
            pltpu.VMEM((_N, _HQ), jnp.float32),
            pltpu.VMEM((2, _K), jnp.int32),
            pltpu.VMEM((2, _K), jnp.int32),
            pltpu.VMEM((_K, _DQ), jnp.float32),
            pltpu.VMEM((_K, _DQ), jnp.float32),
            pltpu.VMEM((_K, 48), jnp.float32),
            pltpu.VMEM((_K, 48), jnp.float32),
            pltpu.VMEM_SHARED((_NR, 48), jnp.float32),
            pltpu.SemaphoreType.DMA,
            pltpu.SemaphoreType.DMA,
            pltpu.SemaphoreType.DMA,
            pltpu.SemaphoreType.DMA,
        ],
    )


def _proj_body(x_ref, wqt, bq, wkt, bk, wvt, bv, mh, p,
               scores_ref, vh_ref, rawi_ref, deni_ref):
    xb = x_ref[...]
    q = jnp.dot(xb, wqt[...], preferred_element_type=jnp.float32) + bq[...]
    k = jnp.dot(xb, wkt[...], preferred_element_type=jnp.float32) + bk[...]
    v = jnp.dot(xb, wvt[...], preferred_element_type=jnp.float32) + bv[...]
    s = jnp.dot(q * k, mh[...], preferred_element_type=jnp.float32) * 0.25
    scores_ref[...] = s
    vh_ref[...] = v
    e0 = jnp.exp(jnp.maximum(2.0 * s, 0.4 * s))
    deni_ref[...] = e0
    rawi_ref[...] = jnp.dot(e0, p[...], preferred_element_type=jnp.float32) * v


def _agg_body(aq0, aq1, aq2, aq3, rawi, deni,
              x_ref, wot, bo, p2m, y_ref, st_ref):
    i = pl.program_id(0)
    f32 = jnp.float32
    a0, a1, a2, a3 = aq0[...], aq1[...], aq2[...], aq3[...]
    den_all = deni[...] + jnp.concatenate(
        [a0[:, _DQ:_DQ + _HQ], a1[:, _DQ:_DQ + _HQ],
         a2[:, _DQ:_DQ + _HQ], a3[:, _DQ:_DQ + _HQ]], axis=1)
    denb = jnp.dot(den_all, p2m[...], preferred_element_type=f32) + 1e-16
    raw = jnp.concatenate(
        [a0[:, :_DQ], a1[:, :_DQ], a2[:, :_DQ], a3[:, :_DQ]], axis=1)
    t = (raw + rawi[...]) / denb
    y = jnp.dot(t, wot[...], preferred_element_type=f32)
    y = y + bo[...] + x_ref[...]
    y_ref[...] = y

    @pl.when(i == 0)
    def _():
        st_ref[...] = jnp.zeros_like(st_ref)

    st_ref[0:1, :] += jnp.sum(y, axis=0, keepdims=True)
    st_ref[1:2, :] += jnp.sum(y * y, axis=0, keepdims=True)


def _ffn_body(y_ref, st_ref, g1, be1, w1t, b1, w2t, b2, f_ref, st2_ref):
    i = pl.program_id(0)
    mean = st_ref[0:1, :] * (1.0 / _N)
    var = st_ref[1:2, :] * (1.0 / _N) - mean * mean
    h = (y_ref[...] - mean) * lax.rsqrt(var + 1e-5) * g1[...] + be1[...]
    t = jnp.dot(h, w1t[...], preferred_element_type=jnp.float32) + b1[...]
    t = jnp.maximum(t, 0.0)
    f = jnp.dot(t, w2t[...], preferred_element_type=jnp.float32) + b2[...] + h
    f_ref[...] = f

    @pl.when(i == 0)
    def _():
        st2_ref[...] = jnp.zeros_like(st2_ref)

    st2_ref[0:1, :] += jnp.sum(f, axis=0, keepdims=True)
    st2_ref[1:2, :] += jnp.sum(f * f, axis=0, keepdims=True)


def _bn2_body(f_ref, st2_ref, g2, be2, out_ref):
    mean = st2_ref[0:1, :] * (1.0 / _N)
    var = st2_ref[1:2, :] * (1.0 / _N) - mean * mean
    out_ref[...] = ((f_ref[...] - mean) * lax.rsqrt(var + 1e-5)
                    * g2[...] + be2[...])


def _full(spec):
    return pl.BlockSpec(spec, lambda i: tuple(0 for _ in spec))


def _rows(w):
    return pl.BlockSpec((_BLK, w), lambda i: (i, 0))


def kernel(x, edge_index, Wq, bq, Wk, bk, Wv, bv, Wo, bo,
           W1, b1, W2, b2, g1, be1, g2, be2):
    f32 = jnp.float32
    j = jnp.arange(_D)
    hd = _HQ * (j // _DQ) + j % _HQ
    perm = hd * _C + (j % _DQ) // _HQ
    hsel = jnp.arange(_H)
    mh = (j[:, None] // _C == hsel[None, :]).astype(f32)
    p2 = (hd[None, :] == hsel[:, None]).astype(f32)

    wqt = Wq.T
    wkt = Wk.T
    wvt = Wv.T[:, perm]
    bvp = bv[perm].reshape(1, _D)
    wot = Wo[:, perm].T
    bq2 = bq.reshape(1, _D)
    bk2 = bk.reshape(1, _D)
    bo2 = bo.reshape(1, _D)
    b12 = b1.reshape(1, _FF)
    b22 = b2.reshape(1, _D)
    g12 = g1.reshape(1, _D)
    be12 = be1.reshape(1, _D)
    g22 = g2.reshape(1, _D)
    be22 = be2.reshape(1, _D)

    scores, vh, rawi, deni = pl.pallas_call(
        _proj_body,
        grid=(_NBLK,),
        in_specs=[
            _rows(_D),
            _full((_D, _D)), _full((1, _D)),
            _full((_D, _D)), _full((1, _D)),
            _full((_D, _D)), _full((1, _D)),
            _full((_D, _H)), _full((_H, _D)),
        ],
        out_specs=[_rows(_H), _rows(_D), _rows(_D), _rows(_H)],
        out_shape=[
            jax.ShapeDtypeStruct((_N, _H), f32),
            jax.ShapeDtypeStruct((_N, _D), f32),
            jax.ShapeDtypeStruct((_N, _D), f32),
            jax.ShapeDtypeStruct((_N, _H), f32),
        ],
    )(x, wqt, bq2, wkt, bk2, wvt, bvp, mh, p2)

    zacc = jnp.zeros((_NR, 48), f32)
    s4 = jnp.transpose(scores.reshape(_N, 4, _HQ), (1, 0, 2))
    src = edge_index[0]
    dst = edge_index[1]
    deff = jnp.where(src == dst, jnp.int32(_N), dst)
    epack = jnp.stack(
        [src.reshape(-1, _K), deff.reshape(-1, _K)], axis=1)
    acc = _sc_edge_call()(
        s4, epack,
        vh[:, 0 * _DQ:1 * _DQ], vh[:, 1 * _DQ:2 * _DQ],
        vh[:, 2 * _DQ:3 * _DQ], vh[:, 3 * _DQ:4 * _DQ], zacc)

    y, st = pl.pallas_call(
        _agg_body,
        grid=(_NBLK,),
        in_specs=[
            _rows(48), _rows(48), _rows(48), _rows(48),
            _rows(_D), _rows(_H),
            _rows(_D), _full((_D, _D)), _full((1, _D)), _full((_H, _D)),
        ],
        out_specs=[_rows(_D), _full((8, _D))],
        out_shape=[
            jax.ShapeDtypeStruct((_N, _D), f32),
            jax.ShapeDtypeStruct((8, _D), f32),
        ],
    )(acc[0, :_N], acc[1, :_N], acc[2, :_N], acc[3, :_N], rawi, deni,
      x, wot, bo2, p2)

    f, st2 = pl.pallas_call(
        _ffn_body,
        grid=(_NBLK,),
        in_specs=[
            _rows(_D), _full((8, _D)), _full((1, _D)), _full((1, _D)),
            _full((_D, _FF)), _full((1, _FF)),
            _full((_FF, _D)), _full((1, _D)),
        ],
        out_specs=[_rows(_D), _full((8, _D))],
        out_shape=[
            jax.ShapeDtypeStruct((_N, _D), f32),
            jax.ShapeDtypeStruct((8, _D), f32),
        ],
    )(y, st, g12, be12, W1.T, b12, W2.T, b22)

    out = pl.pallas_call(
        _bn2_body,
        grid=(_NBLK,),
        in_specs=[_rows(_D), _full((8, _D)), _full((1, _D)), _full((1, _D))],
        out_specs=_rows(_D),
        out_shape=jax.ShapeDtypeStruct((_N, _D), f32),
    )(f, st2, g22, be22)

    return out
